# baseline (speedup 1.0000x reference)
import functools

import jax
import jax.numpy as jnp
from jax import lax
from jax.experimental import pallas as pl
from jax.experimental.pallas import tpu as pltpu
from jax.experimental.pallas import tpu_sc as plsc


def _gather_embeds(inputs, emb_table):
    B = inputs.shape[0]
    V, D = emb_table.shape
    info = plsc.get_sparse_core_info()
    nc, ns = info.num_cores, info.num_subcores
    nw = nc * ns
    b_per_w = B // nw
    mesh = plsc.VectorSubcoreMesh(core_axis_name="c", subcore_axis_name="s")

    @functools.partial(
        pl.kernel,
        mesh=mesh,
        out_type=jax.ShapeDtypeStruct((B, D), jnp.float32),
        scratch_types=[
            pltpu.VMEM((b_per_w,), jnp.int32),
            pltpu.VMEM((b_per_w, D), jnp.float32),
            pltpu.SemaphoreType.DMA,
        ],
    )
    def gather_kernel(idx_hbm, table_hbm, out_hbm, idx_v, rows_v, sem):
        wid = lax.axis_index("s") * nc + lax.axis_index("c")
        base = wid * b_per_w
        pltpu.sync_copy(idx_hbm.at[pl.ds(base, b_per_w)], idx_v)
        pltpu.async_copy(table_hbm.at[idx_v], rows_v, sem).wait()
        pltpu.sync_copy(rows_v, out_hbm.at[pl.ds(base, b_per_w)])

    return gather_kernel(inputs, emb_table)


def _projection(embeds, lin_w, lin_b, tile_n=2048):
    B, D = embeds.shape
    V = lin_w.shape[0]
    bias2d = lin_b.reshape(1, V)

    def body(e_ref, w_ref, b_ref, o_ref):
        acc = lax.dot_general(
            e_ref[...].astype(jnp.bfloat16), w_ref[...].astype(jnp.bfloat16),
            (((1,), (1,)), ((), ())),
            preferred_element_type=jnp.float32,
        )
        o_ref[...] = acc + b_ref[...]

    return pl.pallas_call(
        body,
        grid=(pl.cdiv(V, tile_n),),
        in_specs=[
            pl.BlockSpec((B, D), lambda j: (0, 0)),
            pl.BlockSpec((tile_n, D), lambda j: (j, 0)),
            pl.BlockSpec((1, tile_n), lambda j: (0, j)),
        ],
        out_specs=pl.BlockSpec((B, tile_n), lambda j: (0, j)),
        out_shape=jax.ShapeDtypeStruct((B, V), jnp.float32),
    )(embeds, lin_w, bias2d)


def kernel(inputs, emb_table, lin_w, lin_b):
    embeds = jnp.take(emb_table, inputs, axis=0)
    return _projection(embeds, lin_w, lin_b)


# speedup vs baseline: 3.1387x; 3.1387x over previous
import functools

import jax
import jax.numpy as jnp
from jax import lax
from jax.experimental import pallas as pl
from jax.experimental.pallas import tpu as pltpu
from jax.experimental.pallas import tpu_sc as plsc


def _gather_embeds(inputs, emb_table):
    B = inputs.shape[0]
    V, D = emb_table.shape
    info = plsc.get_sparse_core_info()
    nc, ns = info.num_cores, info.num_subcores
    nw = nc * ns
    b_per_w = B // nw
    mesh = plsc.VectorSubcoreMesh(core_axis_name="c", subcore_axis_name="s")

    @functools.partial(
        pl.kernel,
        mesh=mesh,
        out_type=jax.ShapeDtypeStruct((B, D), jnp.float32),
        scratch_types=[
            pltpu.VMEM((b_per_w,), jnp.int32),
            pltpu.VMEM((b_per_w, D), jnp.float32),
            pltpu.SemaphoreType.DMA,
        ],
    )
    def gather_kernel(idx_hbm, table_hbm, out_hbm, idx_v, rows_v, sem):
        wid = lax.axis_index("s") * nc + lax.axis_index("c")
        base = wid * b_per_w
        pltpu.sync_copy(idx_hbm.at[pl.ds(base, b_per_w)], idx_v)
        pltpu.async_copy(table_hbm.at[idx_v], rows_v, sem).wait()
        pltpu.sync_copy(rows_v, out_hbm.at[pl.ds(base, b_per_w)])

    return gather_kernel(inputs, emb_table)


def _projection_t(embeds, lin_w, lin_b, tile_n=2048):
    B, D = embeds.shape
    V = lin_w.shape[0]
    bias2d = lin_b.reshape(1, V)

    def body(e_ref, w_ref, b_ref, o_ref):
        acc = lax.dot_general(
            w_ref[...].astype(jnp.bfloat16), e_ref[...].astype(jnp.bfloat16),
            (((1,), (1,)), ((), ())),
            preferred_element_type=jnp.float32,
        )
        ones_row = jnp.ones((1, B), jnp.bfloat16)
        acc += lax.dot_general(
            b_ref[...].astype(jnp.bfloat16), ones_row,
            (((0,), (0,)), ((), ())),
            preferred_element_type=jnp.float32,
        )
        o_ref[...] = acc

    return pl.pallas_call(
        body,
        grid=(pl.cdiv(V, tile_n),),
        in_specs=[
            pl.BlockSpec((B, D), lambda j: (0, 0)),
            pl.BlockSpec((tile_n, D), lambda j: (j, 0)),
            pl.BlockSpec((1, tile_n), lambda j: (0, j)),
        ],
        out_specs=pl.BlockSpec((tile_n, B), lambda j: (j, 0)),
        out_shape=jax.ShapeDtypeStruct((V, B), jnp.float32),
    )(embeds, lin_w, bias2d)


def kernel(inputs, emb_table, lin_w, lin_b):
    embeds = _gather_embeds(inputs.astype(jnp.int32), emb_table)
    return _projection_t(embeds, lin_w, lin_b).T


# speedup vs baseline: 3.2093x; 1.0225x over previous
import functools

import jax
import jax.numpy as jnp
from jax import lax
from jax.experimental import pallas as pl
from jax.experimental.pallas import tpu as pltpu
from jax.experimental.pallas import tpu_sc as plsc


def _gather_embeds(inputs, emb_table):
    B = inputs.shape[0]
    V, D = emb_table.shape
    info = plsc.get_sparse_core_info()
    nc, ns = info.num_cores, info.num_subcores
    nw = nc * ns
    b_per_w = B // nw
    mesh = plsc.VectorSubcoreMesh(core_axis_name="c", subcore_axis_name="s")

    @functools.partial(
        pl.kernel,
        mesh=mesh,
        out_type=jax.ShapeDtypeStruct((B, D), jnp.float32),
        scratch_types=[
            pltpu.VMEM((b_per_w,), jnp.int32),
            pltpu.VMEM((b_per_w, D), jnp.float32),
            pltpu.SemaphoreType.DMA,
        ],
    )
    def gather_kernel(idx_hbm, table_hbm, out_hbm, idx_v, rows_v, sem):
        wid = lax.axis_index("s") * nc + lax.axis_index("c")
        base = wid * b_per_w
        pltpu.sync_copy(idx_hbm.at[pl.ds(base, b_per_w)], idx_v)
        pltpu.async_copy(table_hbm.at[idx_v], rows_v, sem).wait()
        pltpu.sync_copy(rows_v, out_hbm.at[pl.ds(base, b_per_w)])

    return gather_kernel(inputs, emb_table)


def _projection_t(embeds, lin_w, lin_b, tile_n=4096):
    B, D = embeds.shape
    V = lin_w.shape[0]
    bias2d = lin_b.reshape(1, V)

    def body(e_ref, w_ref, b_ref, o_ref):
        acc = lax.dot_general(
            w_ref[...].astype(jnp.bfloat16), e_ref[...].astype(jnp.bfloat16),
            (((1,), (1,)), ((), ())),
            preferred_element_type=jnp.float32,
        )
        ones_row = jnp.ones((1, B), jnp.bfloat16)
        acc += lax.dot_general(
            b_ref[...].astype(jnp.bfloat16), ones_row,
            (((0,), (0,)), ((), ())),
            preferred_element_type=jnp.float32,
        )
        o_ref[...] = acc

    return pl.pallas_call(
        body,
        grid=(pl.cdiv(V, tile_n),),
        in_specs=[
            pl.BlockSpec((B, D), lambda j: (0, 0)),
            pl.BlockSpec((tile_n, D), lambda j: (j, 0)),
            pl.BlockSpec((1, tile_n), lambda j: (0, j)),
        ],
        out_specs=pl.BlockSpec((tile_n, B), lambda j: (j, 0)),
        out_shape=jax.ShapeDtypeStruct((V, B), jnp.float32),
    )(embeds, lin_w, bias2d)


def kernel(inputs, emb_table, lin_w, lin_b):
    embeds = _gather_embeds(inputs.astype(jnp.int32), emb_table)
    return _projection_t(embeds, lin_w, lin_b).T


# speedup vs baseline: 3.2193x; 1.0031x over previous
import functools

import jax
import jax.numpy as jnp
from jax import lax
from jax.experimental import pallas as pl
from jax.experimental.pallas import tpu as pltpu
from jax.experimental.pallas import tpu_sc as plsc


def _gather_embeds(inputs, emb_table):
    B = inputs.shape[0]
    V, D = emb_table.shape
    info = plsc.get_sparse_core_info()
    nc, ns = info.num_cores, info.num_subcores
    nw = nc * ns
    b_per_w = B // nw
    mesh = plsc.VectorSubcoreMesh(core_axis_name="c", subcore_axis_name="s")

    @functools.partial(
        pl.kernel,
        mesh=mesh,
        out_type=jax.ShapeDtypeStruct((B, D), jnp.float32),
        scratch_types=[
            pltpu.VMEM((b_per_w,), jnp.int32),
            pltpu.VMEM((b_per_w, D), jnp.float32),
            pltpu.SemaphoreType.DMA,
        ],
    )
    def gather_kernel(idx_hbm, table_hbm, out_hbm, idx_v, rows_v, sem):
        wid = lax.axis_index("s") * nc + lax.axis_index("c")
        base = wid * b_per_w
        pltpu.sync_copy(idx_hbm.at[pl.ds(base, b_per_w)], idx_v)
        pltpu.async_copy(table_hbm.at[idx_v], rows_v, sem).wait()
        pltpu.sync_copy(rows_v, out_hbm.at[pl.ds(base, b_per_w)])

    return gather_kernel(inputs, emb_table)


def _projection_t(embeds, lin_w, lin_b, tile_n=4096):
    B, D = embeds.shape
    V = lin_w.shape[0]

    def body(e_ref, w_ref, b_ref, o_ref):
        acc = lax.dot_general(
            w_ref[...].astype(jnp.bfloat16), e_ref[...].astype(jnp.bfloat16),
            (((1,), (1,)), ((), ())),
            preferred_element_type=jnp.float32,
        )
        ones_row = jnp.ones((1, B), jnp.bfloat16)
        bias_row = b_ref[...].reshape(1, tile_n)
        acc += lax.dot_general(
            bias_row.astype(jnp.bfloat16), ones_row,
            (((0,), (0,)), ((), ())),
            preferred_element_type=jnp.float32,
        )
        o_ref[...] = acc

    return pl.pallas_call(
        body,
        grid=(pl.cdiv(V, tile_n),),
        in_specs=[
            pl.BlockSpec((B, D), lambda j: (0, 0)),
            pl.BlockSpec((tile_n, D), lambda j: (j, 0)),
            pl.BlockSpec((tile_n,), lambda j: (j,)),
        ],
        out_specs=pl.BlockSpec((tile_n, B), lambda j: (j, 0)),
        out_shape=jax.ShapeDtypeStruct((V, B), jnp.float32),
    )(embeds, lin_w, lin_b)


def kernel(inputs, emb_table, lin_w, lin_b):
    embeds = _gather_embeds(inputs.astype(jnp.int32), emb_table)
    return _projection_t(embeds, lin_w, lin_b).T


# speedup vs baseline: 3.2320x; 1.0039x over previous
import functools

import jax
import jax.numpy as jnp
from jax import lax
from jax.experimental import pallas as pl
from jax.experimental.pallas import tpu as pltpu
from jax.experimental.pallas import tpu_sc as plsc


def _gather_embeds(inputs, emb_table):
    B = inputs.shape[0]
    V, D = emb_table.shape
    info = plsc.get_sparse_core_info()
    nc, ns = 1, info.num_subcores
    nw = nc * ns
    b_per_w = B // nw
    mesh = plsc.VectorSubcoreMesh(
        core_axis_name="c", subcore_axis_name="s", num_cores=nc)

    @functools.partial(
        pl.kernel,
        mesh=mesh,
        out_type=jax.ShapeDtypeStruct((B, D), jnp.float32),
        scratch_types=[
            pltpu.VMEM((b_per_w,), jnp.int32),
            pltpu.VMEM((b_per_w, D), jnp.float32),
            pltpu.SemaphoreType.DMA,
        ],
    )
    def gather_kernel(idx_hbm, table_hbm, out_hbm, idx_v, rows_v, sem):
        wid = lax.axis_index("s") * nc + lax.axis_index("c")
        base = wid * b_per_w
        pltpu.sync_copy(idx_hbm.at[pl.ds(base, b_per_w)], idx_v)
        pltpu.async_copy(table_hbm.at[idx_v], rows_v, sem).wait()
        pltpu.sync_copy(rows_v, out_hbm.at[pl.ds(base, b_per_w)])

    return gather_kernel(inputs, emb_table)


def _projection_t(embeds, lin_w, lin_b, tile_n=4096):
    B, D = embeds.shape
    V = lin_w.shape[0]

    def body(e_ref, w_ref, b_ref, o_ref):
        acc = lax.dot_general(
            w_ref[...].astype(jnp.bfloat16), e_ref[...].astype(jnp.bfloat16),
            (((1,), (1,)), ((), ())),
            preferred_element_type=jnp.float32,
        )
        ones_row = jnp.ones((1, B), jnp.bfloat16)
        bias_row = b_ref[...].reshape(1, tile_n)
        acc += lax.dot_general(
            bias_row.astype(jnp.bfloat16), ones_row,
            (((0,), (0,)), ((), ())),
            preferred_element_type=jnp.float32,
        )
        o_ref[...] = acc

    return pl.pallas_call(
        body,
        grid=(pl.cdiv(V, tile_n),),
        in_specs=[
            pl.BlockSpec((B, D), lambda j: (0, 0)),
            pl.BlockSpec((tile_n, D), lambda j: (j, 0)),
            pl.BlockSpec((tile_n,), lambda j: (j,)),
        ],
        out_specs=pl.BlockSpec((tile_n, B), lambda j: (j, 0)),
        out_shape=jax.ShapeDtypeStruct((V, B), jnp.float32),
    )(embeds, lin_w, lin_b)


def kernel(inputs, emb_table, lin_w, lin_b):
    embeds = _gather_embeds(inputs.astype(jnp.int32), emb_table)
    return _projection_t(embeds, lin_w, lin_b).T


# speedup vs baseline: 3.2521x; 1.0062x over previous
import functools

import jax
import jax.numpy as jnp
from jax import lax
from jax.experimental import pallas as pl
from jax.experimental.pallas import tpu as pltpu
from jax.experimental.pallas import tpu_sc as plsc


def _gather_embeds(inputs, emb_table):
    B = inputs.shape[0]
    V, D = emb_table.shape
    info = plsc.get_sparse_core_info()
    nc, ns = 1, info.num_subcores
    nw = nc * ns
    b_per_w = B // nw
    mesh = plsc.VectorSubcoreMesh(
        core_axis_name="c", subcore_axis_name="s", num_cores=nc)

    @functools.partial(
        pl.kernel,
        mesh=mesh,
        out_type=jax.ShapeDtypeStruct((B, D), jnp.float32),
        scratch_types=[
            pltpu.VMEM((b_per_w,), jnp.int32),
            pltpu.VMEM((b_per_w, D), jnp.float32),
            pltpu.SemaphoreType.DMA,
        ],
    )
    def gather_kernel(idx_hbm, table_hbm, out_hbm, idx_v, rows_v, sem):
        wid = lax.axis_index("s") * nc + lax.axis_index("c")
        base = wid * b_per_w
        pltpu.sync_copy(idx_hbm.at[pl.ds(base, b_per_w)], idx_v)
        pltpu.async_copy(table_hbm.at[idx_v], rows_v, sem).wait()
        pltpu.sync_copy(rows_v, out_hbm.at[pl.ds(base, b_per_w)])

    return gather_kernel(inputs, emb_table)


def _projection_t(embeds, lin_w, lin_b, tile_n=5120):
    B, D = embeds.shape
    V = lin_w.shape[0]

    def body(e_ref, w_ref, b_ref, o_ref):
        acc = lax.dot_general(
            w_ref[...].astype(jnp.bfloat16), e_ref[...].astype(jnp.bfloat16),
            (((1,), (1,)), ((), ())),
            preferred_element_type=jnp.float32,
        )
        ones_row = jnp.ones((1, B), jnp.bfloat16)
        bias_row = b_ref[...].reshape(1, tile_n)
        acc += lax.dot_general(
            bias_row.astype(jnp.bfloat16), ones_row,
            (((0,), (0,)), ((), ())),
            preferred_element_type=jnp.float32,
        )
        o_ref[...] = acc

    return pl.pallas_call(
        body,
        grid=(pl.cdiv(V, tile_n),),
        in_specs=[
            pl.BlockSpec((B, D), lambda j: (0, 0)),
            pl.BlockSpec((tile_n, D), lambda j: (j, 0)),
            pl.BlockSpec((tile_n,), lambda j: (j,)),
        ],
        out_specs=pl.BlockSpec((tile_n, B), lambda j: (j, 0)),
        out_shape=jax.ShapeDtypeStruct((V, B), jnp.float32),
    )(embeds, lin_w, lin_b)


def kernel(inputs, emb_table, lin_w, lin_b):
    embeds = _gather_embeds(inputs.astype(jnp.int32), emb_table)
    return _projection_t(embeds, lin_w, lin_b).T
